# Initial kernel scaffold; baseline (speedup 1.0000x reference)
import jax
import jax.numpy as jnp
from jax.experimental import pallas as pl


def kernel(user_id, question_id, theta_table, a_table, b_table, c_table):
    raise NotImplementedError("write your pallas kernel here")


# speedup vs baseline: 1.5754x; 1.5754x over previous
import functools

import jax
import jax.numpy as jnp
from jax import lax
from jax.experimental import pallas as pl
from jax.experimental.pallas import tpu as pltpu
from jax.experimental.pallas import tpu_sc as plsc

_BATCH = 16384
_NUM_CORES = 2
_NUM_SUBCORES = 16
_NUM_WORKERS = _NUM_CORES * _NUM_SUBCORES
_CHUNK = _BATCH // _NUM_WORKERS
_LANES = 16
_VALUE_RANGE = 8.0
_A_RANGE = 4.0
_D = 1.702


def _sigmoid(x):
    return 1.0 / (1.0 + jnp.exp(-x))


def _irt_body(user_id, question_id, theta_t, a_t, b_t, c_t, out,
              idx_u, idx_q, tv, av, bv, cv, ov,
              sem_t, sem_a, sem_b, sem_c):
    wid = lax.axis_index("s") * _NUM_CORES + lax.axis_index("c")
    base = wid * _CHUNK

    pltpu.sync_copy(user_id.at[pl.ds(base, _CHUNK)], idx_u)
    pltpu.sync_copy(question_id.at[pl.ds(base, _CHUNK)], idx_q)

    cp_t = pltpu.async_copy(theta_t.at[idx_u], tv, sem_t)
    cp_a = pltpu.async_copy(a_t.at[idx_q], av, sem_a)
    cp_b = pltpu.async_copy(b_t.at[idx_q], bv, sem_b)
    cp_c = pltpu.async_copy(c_t.at[idx_q], cv, sem_c)
    cp_t.wait()
    cp_a.wait()
    cp_b.wait()
    cp_c.wait()

    for i in range(_CHUNK // _LANES):
        off = i * _LANES
        th = _VALUE_RANGE * (_sigmoid(tv[pl.ds(off, _LANES)]) - 0.5)
        bb = _VALUE_RANGE * (_sigmoid(bv[pl.ds(off, _LANES)]) - 0.5)
        aa = _A_RANGE * _sigmoid(av[pl.ds(off, _LANES)])
        cc = _sigmoid(cv[pl.ds(off, _LANES)])
        ov[pl.ds(off, _LANES)] = cc + (1.0 - cc) * _sigmoid(_D * aa * (th - bb))

    pltpu.sync_copy(ov, out.at[pl.ds(base, _CHUNK)])


@jax.jit
def _irt_sc(user_id, question_id, theta_t, a_t, b_t, c_t):
    mesh = plsc.VectorSubcoreMesh(core_axis_name="c", subcore_axis_name="s")
    f = functools.partial(
        pl.kernel,
        mesh=mesh,
        out_type=jax.ShapeDtypeStruct((_BATCH,), jnp.float32),
        scratch_types=[
            pltpu.VMEM((_CHUNK,), jnp.int32),
            pltpu.VMEM((_CHUNK,), jnp.int32),
            pltpu.VMEM((_CHUNK,), jnp.float32),
            pltpu.VMEM((_CHUNK,), jnp.float32),
            pltpu.VMEM((_CHUNK,), jnp.float32),
            pltpu.VMEM((_CHUNK,), jnp.float32),
            pltpu.VMEM((_CHUNK,), jnp.float32),
            pltpu.SemaphoreType.DMA,
            pltpu.SemaphoreType.DMA,
            pltpu.SemaphoreType.DMA,
            pltpu.SemaphoreType.DMA,
        ],
    )(_irt_body)
    return f(user_id, question_id, theta_t, a_t, b_t, c_t)


def kernel(user_id, question_id, theta_table, a_table, b_table, c_table):
    return _irt_sc(
        user_id.astype(jnp.int32),
        question_id.astype(jnp.int32),
        theta_table.reshape(-1),
        a_table.reshape(-1),
        b_table.reshape(-1),
        c_table.reshape(-1),
    )


# speedup vs baseline: 1.6608x; 1.0542x over previous
import functools

import jax
import jax.numpy as jnp
from jax import lax
from jax.experimental import pallas as pl
from jax.experimental.pallas import tpu as pltpu
from jax.experimental.pallas import tpu_sc as plsc

_BATCH = 16384
_NUM_CORES = 2
_NUM_SUBCORES = 16
_NUM_WORKERS = _NUM_CORES * _NUM_SUBCORES
_CHUNK = _BATCH // _NUM_WORKERS
_HALF = _CHUNK // 2
_LANES = 16
_SCALE = 1.702 * 4.0 * 8.0


def _irt_body(user_id, question_id, theta_t, a_t, b_t, c_t, out,
              idx_u, idx_q, tv, av, bv, cv, ov,
              sem_iu, sem_iq,
              s_t0, s_a0, s_b0, s_c0, s_t1, s_a1, s_b1, s_c1):
    wid = lax.axis_index("s") * _NUM_CORES + lax.axis_index("c")
    base = wid * _CHUNK

    cu = pltpu.async_copy(user_id.at[pl.ds(base, _CHUNK)], idx_u, sem_iu)
    cq = pltpu.async_copy(question_id.at[pl.ds(base, _CHUNK)], idx_q, sem_iq)
    cu.wait()
    cq.wait()

    sems = ((s_t0, s_a0, s_b0, s_c0), (s_t1, s_a1, s_b1, s_c1))
    cps = []
    for h in range(2):
        off = h * _HALF
        iu = idx_u.at[pl.ds(off, _HALF)]
        iq = idx_q.at[pl.ds(off, _HALF)]
        st, sa, sb, sc = sems[h]
        cps.append((
            pltpu.async_copy(theta_t.at[iu], tv.at[pl.ds(off, _HALF)], st),
            pltpu.async_copy(a_t.at[iq], av.at[pl.ds(off, _HALF)], sa),
            pltpu.async_copy(b_t.at[iq], bv.at[pl.ds(off, _HALF)], sb),
            pltpu.async_copy(c_t.at[iq], cv.at[pl.ds(off, _HALF)], sc),
        ))

    for h in range(2):
        for cp in cps[h]:
            cp.wait()
        for i in range(_HALF // _LANES):
            off = h * _HALF + i * _LANES
            et = jnp.exp(-tv[pl.ds(off, _LANES)])
            ea = jnp.exp(-av[pl.ds(off, _LANES)])
            eb = jnp.exp(-bv[pl.ds(off, _LANES)])
            ec = jnp.exp(-cv[pl.ds(off, _LANES)])
            z = _SCALE * (eb - et) / ((1.0 + ea) * (1.0 + et) * (1.0 + eb))
            ez = jnp.exp(-z)
            ov[pl.ds(off, _LANES)] = (1.0 + ez + ec) / ((1.0 + ez) * (1.0 + ec))

    pltpu.sync_copy(ov, out.at[pl.ds(base, _CHUNK)])


@jax.jit
def _irt_sc(user_id, question_id, theta_t, a_t, b_t, c_t):
    mesh = plsc.VectorSubcoreMesh(core_axis_name="c", subcore_axis_name="s")
    f = functools.partial(
        pl.kernel,
        mesh=mesh,
        out_type=jax.ShapeDtypeStruct((_BATCH,), jnp.float32),
        scratch_types=[
            pltpu.VMEM((_CHUNK,), jnp.int32),
            pltpu.VMEM((_CHUNK,), jnp.int32),
            pltpu.VMEM((_CHUNK,), jnp.float32),
            pltpu.VMEM((_CHUNK,), jnp.float32),
            pltpu.VMEM((_CHUNK,), jnp.float32),
            pltpu.VMEM((_CHUNK,), jnp.float32),
            pltpu.VMEM((_CHUNK,), jnp.float32),
        ] + [pltpu.SemaphoreType.DMA] * 10,
    )(_irt_body)
    return f(user_id, question_id, theta_t, a_t, b_t, c_t)


def kernel(user_id, question_id, theta_table, a_table, b_table, c_table):
    return _irt_sc(
        user_id.astype(jnp.int32),
        question_id.astype(jnp.int32),
        theta_table.reshape(-1),
        a_table.reshape(-1),
        b_table.reshape(-1),
        c_table.reshape(-1),
    )


# speedup vs baseline: 1.6965x; 1.0215x over previous
import functools

import jax
import jax.numpy as jnp
from jax import lax
from jax.experimental import pallas as pl
from jax.experimental.pallas import tpu as pltpu
from jax.experimental.pallas import tpu_sc as plsc

_BATCH = 16384
_NUM_CORES = 2
_NUM_SUBCORES = 16
_NUM_WORKERS = _NUM_CORES * _NUM_SUBCORES
_CHUNK = _BATCH // _NUM_WORKERS
_HALF = _CHUNK // 2
_LANES = 16
_SCALE = 1.702 * 4.0 * 8.0


def _irt_body(user_id, question_id, theta_t, a_t, b_t, c_t, out,
              idx_u, idx_q, tv, av, bv, cv, ov,
              sem_iu, sem_iq,
              s_t0, s_a0, s_b0, s_c0, s_t1, s_a1, s_b1, s_c1):
    wid = lax.axis_index("s") * _NUM_CORES + lax.axis_index("c")
    base = wid * _CHUNK

    cu = pltpu.async_copy(user_id.at[pl.ds(base, _CHUNK)], idx_u, sem_iu)
    cq = pltpu.async_copy(question_id.at[pl.ds(base, _CHUNK)], idx_q, sem_iq)
    cu.wait()
    cq.wait()

    sems = ((s_t0, s_a0, s_b0, s_c0), (s_t1, s_a1, s_b1, s_c1))
    cps = []
    for h in range(2):
        off = h * _HALF
        iu = idx_u.at[pl.ds(off, _HALF)]
        iq = idx_q.at[pl.ds(off, _HALF)]
        st, sa, sb, sc = sems[h]
        cps.append((
            pltpu.async_copy(theta_t.at[iu], tv.at[pl.ds(off, _HALF)], st),
            pltpu.async_copy(a_t.at[iq], av.at[pl.ds(off, _HALF)], sa),
            pltpu.async_copy(b_t.at[iq], bv.at[pl.ds(off, _HALF)], sb),
            pltpu.async_copy(c_t.at[iq], cv.at[pl.ds(off, _HALF)], sc),
        ))

    for h in range(2):
        for cp in cps[h]:
            cp.wait()

        def step(i, _, _h=h):
            off = pl.multiple_of(_h * _HALF + i * _LANES, _LANES)
            et = jnp.exp(-tv[pl.ds(off, _LANES)])
            ea = jnp.exp(-av[pl.ds(off, _LANES)])
            eb = jnp.exp(-bv[pl.ds(off, _LANES)])
            ec = jnp.exp(-cv[pl.ds(off, _LANES)])
            z = _SCALE * (eb - et) / ((1.0 + ea) * (1.0 + et) * (1.0 + eb))
            ez = jnp.exp(-z)
            ov[pl.ds(off, _LANES)] = (1.0 + ez + ec) / ((1.0 + ez) * (1.0 + ec))
            return 0

        lax.fori_loop(0, _HALF // _LANES, step, 0, unroll=2)

    pltpu.sync_copy(ov, out.at[pl.ds(base, _CHUNK)])


@jax.jit
def _irt_sc(user_id, question_id, theta_t, a_t, b_t, c_t):
    mesh = plsc.VectorSubcoreMesh(core_axis_name="c", subcore_axis_name="s")
    f = functools.partial(
        pl.kernel,
        mesh=mesh,
        out_type=jax.ShapeDtypeStruct((_BATCH,), jnp.float32),
        scratch_types=[
            pltpu.VMEM((_CHUNK,), jnp.int32),
            pltpu.VMEM((_CHUNK,), jnp.int32),
            pltpu.VMEM((_CHUNK,), jnp.float32),
            pltpu.VMEM((_CHUNK,), jnp.float32),
            pltpu.VMEM((_CHUNK,), jnp.float32),
            pltpu.VMEM((_CHUNK,), jnp.float32),
            pltpu.VMEM((_CHUNK,), jnp.float32),
        ] + [pltpu.SemaphoreType.DMA] * 10,
    )(_irt_body)
    return f(user_id, question_id, theta_t, a_t, b_t, c_t)


def kernel(user_id, question_id, theta_table, a_table, b_table, c_table):
    return _irt_sc(
        user_id.astype(jnp.int32),
        question_id.astype(jnp.int32),
        theta_table.reshape(-1),
        a_table.reshape(-1),
        b_table.reshape(-1),
        c_table.reshape(-1),
    )
